# baseline (speedup 1.0000x reference)
import functools

import jax
import jax.numpy as jnp
from jax import lax
from jax.experimental import pallas as pl
from jax.experimental.pallas import tpu as pltpu
from jax.experimental.pallas import tpu_sc as plsc

_B = 128
_N = 32768
_BLK = 128
_NBLK = _N // _BLK
_CAP = 256
_NC, _NS, _L = 2, 16, 16
_NW = _NC * _NS
_RPW = _B // _NW
_NEWTON = 12


def _blockmax_kernel(z_ref, bm_ref):
    z = z_ref[:, :]
    bm_ref[:, :] = jnp.max(z.reshape(_B, _NBLK, _BLK), axis=-1)


def _scalar_sum(vec):
    s = vec[0]
    for q in range(1, _L):
        s = s + vec[q]
    return s


def _tau_sc_kernel(z2_hbm, bm_hbm, out_hbm, bm_v, idx_v, cand_v, tau_v, sem):
    wid = lax.axis_index("s") * _NC + lax.axis_index("c")
    lane = lax.iota(jnp.int32, _L)

    for j in range((_CAP + _L) // _L):
        idx_v[pl.ds(j * _L, _L)] = jnp.zeros((_L,), jnp.int32)

    def _row(i, taus):
        r = wid * _RPW + i
        pltpu.sync_copy(bm_hbm.at[r], bm_v)

        def _mx(j, acc):
            return jnp.maximum(acc, bm_v[pl.ds(j * _L, _L)])
        acc = lax.fori_loop(0, _NBLK // _L, _mx,
                            jnp.full((_L,), -3.4e38, jnp.float32))
        rmax = acc[0]
        for j in range(1, _L):
            rmax = jnp.maximum(rmax, acc[j])
        t0 = rmax - 1.0

        def _ext(j, cursor):
            bmv = bm_v[pl.ds(j * _L, _L)]
            base = r * _NBLK + j * _L
            for q in range(_L):
                cond = bmv[q] > t0

                @pl.when(jnp.logical_and(cond, cursor < _CAP))
                def _append():
                    idx_v[pl.ds(cursor, _L)] = jnp.full(
                        (_L,), base + q, jnp.int32)
                cursor = jnp.where(cond, cursor + 1, cursor)
            return cursor
        cursor = lax.fori_loop(0, _NBLK // _L, _ext, jnp.zeros((), jnp.int32))
        ncand = jnp.minimum(cursor, _CAP)

        for c in range(_CAP // 32):
            @pl.when(c * 32 < ncand)
            def _gather():
                cp = pltpu.async_copy(
                    z2_hbm.at[idx_v.at[pl.ds(c * 32, 32)]],
                    cand_v.at[pl.ds(c * 32, 32)], sem)
                cp.wait()

        def _newton(_, t):
            def _blk(b, fk):
                facc, kacc = fk
                for l in range(_BLK // _L):
                    v = cand_v[b, pl.ds(l * _L, _L)]
                    d = v - t
                    facc = facc + jnp.maximum(d, 0.0)
                    kacc = kacc + jnp.where(d > 0.0, 1.0, 0.0)
                return facc, kacc
            zero = jnp.zeros((_L,), jnp.float32)
            facc, kacc = lax.fori_loop(0, ncand, _blk, (zero, zero))
            fvec = jnp.broadcast_to(_scalar_sum(facc), (_L,))
            kvec = jnp.broadcast_to(_scalar_sum(kacc), (_L,))
            return t + (fvec - 1.0) / jnp.maximum(kvec, 1.0)
        t0vec = jnp.broadcast_to(t0, (_L,))
        t = lax.fori_loop(0, _NEWTON, _newton, t0vec)

        return jnp.where(lane == i, t, taus)

    taus = lax.fori_loop(0, _RPW, _row, jnp.zeros((_L,), jnp.float32))
    tau_v[pl.ds(0, _L)] = taus
    pltpu.sync_copy(tau_v, out_hbm.at[wid])


def _clip_mean_kernel(z_ref, tau_ref, out_ref):
    z = z_ref[:, :]
    tau = tau_ref[:, :]
    p = jnp.maximum(z - tau, 0.0)
    col_mean = jnp.mean(p, axis=0, keepdims=True)
    out_ref[:, :] = jnp.broadcast_to(col_mean, (_B, _N))


def kernel(z):
    bm = pl.pallas_call(
        _blockmax_kernel,
        out_shape=jax.ShapeDtypeStruct((_B, _NBLK), jnp.float32),
    )(z)

    z2 = z.reshape(_B * _NBLK, _BLK)
    mesh = plsc.VectorSubcoreMesh(core_axis_name="c", subcore_axis_name="s")
    tau_w = pl.kernel(
        _tau_sc_kernel,
        out_type=jax.ShapeDtypeStruct((_NW, _L), jnp.float32),
        mesh=mesh,
        scratch_types=[
            pltpu.VMEM((_NBLK,), jnp.float32),
            pltpu.VMEM((_CAP + _L,), jnp.int32),
            pltpu.VMEM((_CAP, _BLK), jnp.float32),
            pltpu.VMEM((_L,), jnp.float32),
            pltpu.SemaphoreType.DMA,
        ],
    )(z2, bm)

    tau = tau_w[:, :_RPW].reshape(_B, 1)
    return pl.pallas_call(
        _clip_mean_kernel,
        out_shape=jax.ShapeDtypeStruct((_B, _N), z.dtype),
    )(z, tau)


# speedup vs baseline: 1.0429x; 1.0429x over previous
import functools

import jax
import jax.numpy as jnp
from jax import lax
from jax.experimental import pallas as pl
from jax.experimental.pallas import tpu as pltpu
from jax.experimental.pallas import tpu_sc as plsc

_B = 128
_N = 32768
_BLK = 128
_NBLK = _N // _BLK
_CAP = 256
_NC, _NS, _L = 2, 16, 16
_NW = _NC * _NS
_RPW = _B // _NW
_NEWTON = 10


def _blockmax_kernel(z_ref, bm_ref):
    z = z_ref[:, :]
    bm_ref[:, :] = jnp.max(z.reshape(_B, _NBLK, _BLK), axis=-1)


def _scalar_sum(vec):
    s = vec[0]
    for q in range(1, _L):
        s = s + vec[q]
    return s


def _tau_sc_kernel(z2_hbm, bm_hbm, out_hbm, bm_v, idx_v, cand_v, tau_v, sem):
    wid = lax.axis_index("s") * _NC + lax.axis_index("c")
    lane = lax.iota(jnp.int32, _L)

    for j in range((_CAP + _L) // _L):
        idx_v[pl.ds(j * _L, _L)] = jnp.zeros((_L,), jnp.int32)

    def _row(i, taus):
        r = wid * _RPW + i
        pltpu.sync_copy(bm_hbm.at[r], bm_v)

        def _mx(j, acc):
            return jnp.maximum(acc, bm_v[pl.ds(j * _L, _L)])
        acc = lax.fori_loop(0, _NBLK // _L, _mx,
                            jnp.full((_L,), -3.4e38, jnp.float32))
        rmax = acc[0]
        for j in range(1, _L):
            rmax = jnp.maximum(rmax, acc[j])
        t0 = rmax - 1.0

        def _ext(j, cursor):
            bmv = bm_v[pl.ds(j * _L, _L)]
            base = r * _NBLK + j * _L
            for q in range(_L):
                cond = bmv[q] > t0

                @pl.when(jnp.logical_and(cond, cursor < _CAP))
                def _append():
                    idx_v[pl.ds(cursor, _L)] = jnp.full(
                        (_L,), base + q, jnp.int32)
                cursor = jnp.where(cond, cursor + 1, cursor)
            return cursor
        cursor = lax.fori_loop(0, _NBLK // _L, _ext, jnp.zeros((), jnp.int32))
        ncand = jnp.minimum(cursor, _CAP)

        for c in range(_CAP // 32):
            @pl.when(c * 32 < ncand)
            def _gather():
                cp = pltpu.async_copy(
                    z2_hbm.at[idx_v.at[pl.ds(c * 32, 32)]],
                    cand_v.at[pl.ds(c * 32, 32)], sem)
                cp.wait()

        _NSUB = _BLK // _L

        def _newton(_, t):
            def _blk(b, fk):
                faccs, kaccs = fk
                nf, nk = [], []
                for l in range(_NSUB):
                    v = cand_v[b, pl.ds(l * _L, _L)]
                    d = v - t
                    nf.append(faccs[l] + jnp.maximum(d, 0.0))
                    nk.append(kaccs[l] + jnp.where(d > 0.0, 1.0, 0.0))
                return tuple(nf), tuple(nk)
            zero = jnp.zeros((_L,), jnp.float32)
            faccs, kaccs = lax.fori_loop(
                0, ncand, _blk,
                ((zero,) * _NSUB, (zero,) * _NSUB))
            facc, kacc = faccs[0], kaccs[0]
            for l in range(1, _NSUB):
                facc = facc + faccs[l]
                kacc = kacc + kaccs[l]
            fvec = jnp.broadcast_to(_scalar_sum(facc), (_L,))
            kvec = jnp.broadcast_to(_scalar_sum(kacc), (_L,))
            return t + (fvec - 1.0) / jnp.maximum(kvec, 1.0)
        t0vec = jnp.broadcast_to(t0, (_L,))
        t = lax.fori_loop(0, _NEWTON, _newton, t0vec)

        return jnp.where(lane == i, t, taus)

    taus = lax.fori_loop(0, _RPW, _row, jnp.zeros((_L,), jnp.float32))
    tau_v[pl.ds(0, _L)] = taus
    pltpu.sync_copy(tau_v, out_hbm.at[wid])


def _clip_mean_kernel(z_ref, tau_ref, out_ref):
    z = z_ref[:, :]
    tau = tau_ref[:, :]
    p = jnp.maximum(z - tau, 0.0)
    col_mean = jnp.mean(p, axis=0, keepdims=True)
    out_ref[:, :] = jnp.broadcast_to(col_mean, (_B, _N))


def kernel(z):
    bm = pl.pallas_call(
        _blockmax_kernel,
        out_shape=jax.ShapeDtypeStruct((_B, _NBLK), jnp.float32),
    )(z)

    z2 = z.reshape(_B * _NBLK, _BLK)
    mesh = plsc.VectorSubcoreMesh(core_axis_name="c", subcore_axis_name="s")
    tau_w = pl.kernel(
        _tau_sc_kernel,
        out_type=jax.ShapeDtypeStruct((_NW, _L), jnp.float32),
        mesh=mesh,
        scratch_types=[
            pltpu.VMEM((_NBLK,), jnp.float32),
            pltpu.VMEM((_CAP + _L,), jnp.int32),
            pltpu.VMEM((_CAP, _BLK), jnp.float32),
            pltpu.VMEM((_L,), jnp.float32),
            pltpu.SemaphoreType.DMA,
        ],
    )(z2, bm)

    tau = tau_w[:, :_RPW].reshape(_B, 1)
    return pl.pallas_call(
        _clip_mean_kernel,
        out_shape=jax.ShapeDtypeStruct((_B, _N), z.dtype),
    )(z, tau)
